# baseline (speedup 1.0000x reference)
import functools

import jax
import jax.numpy as jnp
from jax import lax
from jax.experimental import pallas as pl
from jax.experimental.pallas import tpu as pltpu
from jax.experimental.pallas import tpu_sc as plsc

B = 4096
T = 77
D = 768
V = 49408

NC = 2
NS = 16
NW = NC * NS
BPW = B // NW
C = 64
NVREG = D // 16


def _make_embed_kernel():
    mesh = plsc.VectorSubcoreMesh(core_axis_name="c", subcore_axis_name="s")

    @functools.partial(
        pl.kernel,
        out_type=jax.ShapeDtypeStruct((B * T, D), jnp.float32),
        mesh=mesh,
        scratch_types=[
            pltpu.VMEM((C,), jnp.int32),
            pltpu.VMEM((C,), jnp.int32),
            pltpu.VMEM((C,), jnp.int32),
            pltpu.VMEM((C,), jnp.int32),
            pltpu.VMEM((D,), jnp.float32),
            pltpu.VMEM((D,), jnp.float32),
            pltpu.VMEM((C, D), jnp.float32),
            pltpu.VMEM((C, D), jnp.float32),
            pltpu.SemaphoreType.DMA,
            pltpu.SemaphoreType.DMA,
            pltpu.SemaphoreType.DMA,
            pltpu.SemaphoreType.DMA,
        ],
    )
    def embed(ids_hbm, table_hbm, pos_hbm, out_hbm,
              idx0, idx1, oidx0, oidx1, pos0, pos1, rows0, rows1,
              g0, g1, o0, o1):
        wid = lax.axis_index("s") * NC + lax.axis_index("c")
        b_base = wid * BPW

        def fetch(p, ch, idxb, posb, rowsb, gsem):
            b0 = b_base + ch * C
            pltpu.sync_copy(ids_hbm.at[pl.ds(p * B + b0, C)], idxb)
            pltpu.sync_copy(pos_hbm.at[pl.ds(p * D, D)], posb)
            pltpu.async_copy(table_hbm.at[idxb], rowsb, gsem)

        def process(p, ch, idxb, oib, posb, rowsb, gsem, osem):
            pltpu.make_async_copy(table_hbm.at[idxb], rowsb, gsem).wait()
            b0 = b_base + ch * C
            for g in range(C // 16):
                lane = lax.iota(jnp.int32, 16)
                oib[pl.ds(g * 16, 16)] = (lane + (b0 + g * 16)) * T + p
            def row_body(b, acc):
                for k in range(NVREG):
                    sl = pl.ds(k * 16, 16)
                    plsc.addupdate(rowsb.at[b, sl], posb[sl])
                return acc
            lax.fori_loop(0, C, row_body, 0)
            pltpu.async_copy(rowsb, out_hbm.at[oib], osem)
            @pl.when(p + 1 < T)
            def _():
                pltpu.sync_copy(ids_hbm.at[pl.ds((p + 1) * B + b0, C)], idxb)
                pltpu.sync_copy(pos_hbm.at[pl.ds((p + 1) * D, D)], posb)
                pltpu.make_async_copy(rowsb, out_hbm.at[oib], osem).wait()
                pltpu.async_copy(table_hbm.at[idxb], rowsb, gsem)

        fetch(0, 0, idx0, pos0, rows0, g0)
        fetch(0, 1, idx1, pos1, rows1, g1)

        def trip(p, acc):
            process(p, 0, idx0, oidx0, pos0, rows0, g0, o0)
            process(p, 1, idx1, oidx1, pos1, rows1, g1, o1)
            return acc

        lax.fori_loop(0, T, trip, 0)

        pltpu.make_async_copy(rows0, out_hbm.at[oidx0], o0).wait()
        pltpu.make_async_copy(rows1, out_hbm.at[oidx1], o1).wait()

    return embed


_embed = _make_embed_kernel()


@jax.jit
def kernel(input_ids, table, pos_table):
    ids_t = input_ids.astype(jnp.int32).T.reshape(-1)
    pos_flat = pos_table.reshape(-1)
    out_flat = _embed(ids_t, table, pos_flat)
    return out_flat.reshape(B, T, D)
